# baseline (speedup 1.0000x reference)
import functools

import jax
import jax.numpy as jnp
from jax.experimental import pallas as pl

N = 50000
NEG_SLOPE = 0.2


def _mm_body(x_ref, w_ref, o_ref):
    o_ref[...] = jnp.dot(x_ref[...], w_ref[...],
                         preferred_element_type=jnp.float32)


def _project(h, W):
    n, k = h.shape
    m = W.shape[1]
    blk = 2000
    grid = n // blk
    return pl.pallas_call(
        _mm_body,
        grid=(grid,),
        in_specs=[
            pl.BlockSpec((blk, k), lambda i: (i, 0)),
            pl.BlockSpec((k, m), lambda i: (0, 0)),
        ],
        out_specs=pl.BlockSpec((blk, m), lambda i: (i, 0)),
        out_shape=jax.ShapeDtypeStruct((n, m), jnp.float32),
    )(h, W)


def _layer(h, src, dst, W, attn, H, D, apply_act):
    feat = _project(h, W).reshape(-1, H, D)
    el = feat[src]
    er = feat[dst]
    e = jax.nn.leaky_relu(el + er, negative_slope=NEG_SLOPE)
    logits = jnp.sum(e * attn, axis=-1)
    m = jax.ops.segment_max(logits, dst, num_segments=N)
    ex = jnp.exp(logits - m[dst])
    s = jax.ops.segment_sum(ex, dst, num_segments=N)
    alpha = ex / s[dst]
    msg = el * alpha[..., None]
    rst = jax.ops.segment_sum(msg, dst, num_segments=N)
    if apply_act:
        rst = jax.nn.elu(rst)
    return rst


def kernel(x, edge_index, W0, a0, W1, a1, W2, a2, W3, a3):
    src = edge_index[0]
    dst = edge_index[1]
    h = _layer(x, src, dst, W0, a0, 4, 32, True).reshape(N, 128)
    h = _layer(h, src, dst, W1, a1, 4, 32, True).reshape(N, 128)
    h = _layer(h, src, dst, W2, a2, 4, 32, True).reshape(N, 128)
    logits = _layer(h, src, dst, W3, a3, 4, 8, False).mean(axis=1)
    return logits
